# baseline (speedup 1.0000x reference)
import functools

import jax
import jax.numpy as jnp
from jax import lax
from jax.experimental import pallas as pl
from jax.experimental.pallas import tpu as pltpu
from jax.experimental.pallas import tpu_sc as plsc

HIDDEN = 128
EPS = 1e-12

NC, NS = 2, 16
NW = NC * NS
CHUNK = 80
K = 2
GROUP = K * CHUNK
PH = GROUP // 2

BB = 64
SLICES = 4


def _sc_gather_pack(word_emb, idx3):
    n_chunks = idx3.shape[1]
    n_rows = NW * n_chunks * CHUNK
    per_w = n_chunks * CHUNK
    n_groups = n_chunks // K
    assert n_chunks % K == 0 and n_groups >= 4
    mesh = plsc.VectorSubcoreMesh(core_axis_name="c", subcore_axis_name="s")

    @functools.partial(
        pl.kernel,
        out_type=jax.ShapeDtypeStruct((n_rows // 2, HIDDEN), jnp.uint32),
        mesh=mesh,
        compiler_params=pltpu.CompilerParams(needs_layout_passes=False),
        scratch_types=[
            pltpu.VMEM((n_chunks, CHUNK), jnp.int32),
            pltpu.VMEM((2 * GROUP, HIDDEN), jnp.float32),
            pltpu.VMEM((2 * PH, HIDDEN), jnp.uint32),
            pltpu.SemaphoreType.DMA,
            pltpu.SemaphoreType.DMA,
        ],
    )
    def k(table_hbm, idx_hbm, out_hbm, idx_v, rows_v, pk_v, gsem, wsem):
        wid = lax.axis_index("s") * NC + lax.axis_index("c")
        base_pk = wid * (per_w // 2)

        pltpu.sync_copy(idx_hbm.at[wid], idx_v)

        def fire_group(g, half):
            for t in range(K):
                pltpu.async_copy(
                    table_hbm.at[idx_v.at[g * K + t]],
                    rows_v.at[pl.ds(half * GROUP + t * CHUNK, CHUNK)],
                    gsem,
                )

        def drain_gathers():
            for _ in range(K):
                pltpu.make_async_copy(
                    table_hbm.at[idx_v.at[0]], rows_v.at[pl.ds(0, CHUNK)], gsem
                ).wait()

        def convert(half):
            fb = half * GROUP
            pb = half * PH

            @plsc.parallel_loop(0, PH, 1, unroll=4)
            def _(l):
                for j in range(HIDDEN // 16):
                    ai = plsc.bitcast(rows_v[fb + l, pl.ds(j * 16, 16)],
                                      jnp.uint32)
                    bi = plsc.bitcast(rows_v[fb + PH + l, pl.ds(j * 16, 16)],
                                      jnp.uint32)
                    word = (ai >> 16) | (bi & jnp.uint32(0xFFFF0000))
                    pk_v[pb + l, pl.ds(j * 16, 16)] = word

        def fire_write(g, half):
            pltpu.async_copy(
                pk_v.at[pl.ds(half * PH, PH)],
                out_hbm.at[pl.ds(base_pk + g * PH, PH)],
                wsem,
            )

        def drain_write():
            pltpu.make_async_copy(
                pk_v.at[pl.ds(0, PH)], out_hbm.at[pl.ds(base_pk, PH)], wsem
            ).wait()

        fire_group(0, 0)
        fire_group(1, 1)
        drain_gathers()
        convert(0)
        fire_write(0, 0)
        fire_group(2, 0)
        drain_gathers()
        convert(1)
        fire_write(1, 1)

        def body(g, _):
            half = g % 2
            fire_group(g + 1, 1 - half)
            drain_gathers()
            drain_write()
            convert(half)
            fire_write(g, half)
            return 0

        lax.fori_loop(2, n_groups - 1, body, 0)

        gl = n_groups - 1
        drain_gathers()
        drain_write()
        convert(gl % 2)
        fire_write(gl, gl % 2)
        drain_write()
        drain_write()

    return k(word_emb, idx3)


def _tc_ln_body(tt_ref, pk_ref, pos_ref, td_ref, gm_ref, bt_ref, o_ref):
    bb, s, h = o_ref.shape
    rows = bb * s
    w = pk_ref[...]
    lo = lax.bitcast_convert_type(w << 16, jnp.float32)
    hi = lax.bitcast_convert_type(w & jnp.uint32(0xFFFF0000), jnp.float32)
    nstrip = (rows // 2) // PH
    x = jnp.stack(
        [lo.reshape(nstrip, PH, h), hi.reshape(nstrip, PH, h)], axis=1
    ).reshape(bb, s, h)
    t = tt_ref[...].astype(jnp.float32)[..., None]
    x = x + pos_ref[...][None, :, :] + t * td_ref[...][None, :, :]
    x2 = x.reshape(rows, h)
    ones = jnp.ones((h, h), jnp.bfloat16)
    inv_h = 1.0 / h
    xb = x2.astype(jnp.bfloat16)
    s1 = lax.dot_general(xb, ones, (((1,), (0,)), ((), ())),
                         preferred_element_type=jnp.float32)
    s2 = lax.dot_general(xb * xb, ones, (((1,), (0,)), ((), ())),
                         preferred_element_type=jnp.float32)
    mean = s1 * inv_h
    var = s2 * inv_h - mean * mean
    scale = lax.rsqrt(var + EPS) * gm_ref[...].reshape(1, h)
    o_ref[...] = ((x2 - mean) * scale).reshape(bb, s, h) + bt_ref[...][None, :, :]


def _tc_ln(tt, pk, pos, td, gm, bt, prev, off, B, S):
    H = HIDDEN
    Gh = (2 * pk.shape[0]) // (BB * S)
    pkb = BB * S // 2
    in_specs = [
        pl.BlockSpec((BB, S), lambda i, off=off: (i + off, 0)),
        pl.BlockSpec((pkb, H), lambda i: (i, 0)),
        pl.BlockSpec((S, H), lambda i: (0, 0)),
        pl.BlockSpec((1, H), lambda i: (0, 0)),
        pl.BlockSpec((1, H), lambda i: (0, 0)),
        pl.BlockSpec((1, H), lambda i: (0, 0)),
    ]
    args = [tt, pk, pos, td, gm, bt]
    aliases = {}
    if prev is not None:
        in_specs.append(pl.BlockSpec(memory_space=pl.ANY))
        args.append(prev)
        aliases = {6: 0}

    def body(*refs):
        _tc_ln_body(*refs[:6], refs[-1])

    return pl.pallas_call(
        body,
        out_shape=jax.ShapeDtypeStruct((B, S, H), jnp.float32),
        grid=(Gh,),
        in_specs=in_specs,
        out_specs=pl.BlockSpec((BB, S, H), lambda i, off=off: (i + off, 0, 0)),
        input_output_aliases=aliases,
    )(*args)


def kernel(input_ids, token_type_ids, word_emb, pos_emb, type_emb, ln_gamma, ln_beta):
    B, S = input_ids.shape
    H = HIDDEN
    n = B * S
    Bh = B // SLICES
    nh = n // SLICES
    assert nh % (NW * CHUNK) == 0 and Bh % BB == 0 and (BB * S) % GROUP == 0
    n_chunks = nh // (NW * CHUNK)
    ids = input_ids.reshape(SLICES, NW, n_chunks, CHUNK).astype(jnp.int32)
    tt = token_type_ids.astype(jnp.int32)

    pos = pos_emb[:S] + type_emb[0][None, :]
    td = (type_emb[1] - type_emb[0])[None, :]
    gm = ln_gamma[None, :]
    bt = ln_beta[None, :]

    pks = [_sc_gather_pack(word_emb, ids[s]) for s in range(SLICES)]
    out = None
    for s in range(SLICES):
        out = _tc_ln(tt, pks[s], pos, td, gm, bt, out, s * (Bh // BB), B, S)
    return out


# speedup vs baseline: 1.0561x; 1.0561x over previous
import functools

import jax
import jax.numpy as jnp
from jax import lax
from jax.experimental import pallas as pl
from jax.experimental.pallas import tpu as pltpu
from jax.experimental.pallas import tpu_sc as plsc

HIDDEN = 128
EPS = 1e-12

NC, NS = 2, 16
NW = NC * NS
CHUNK = 80
K = 2
GROUP = K * CHUNK
PH = GROUP // 2

BB = 64
SLICES = 2


def _sc_gather_pack(word_emb, idx3):
    n_chunks = idx3.shape[1]
    n_rows = NW * n_chunks * CHUNK
    per_w = n_chunks * CHUNK
    n_groups = n_chunks // K
    assert n_chunks % K == 0 and n_groups >= 4
    mesh = plsc.VectorSubcoreMesh(core_axis_name="c", subcore_axis_name="s")

    @functools.partial(
        pl.kernel,
        out_type=jax.ShapeDtypeStruct((n_rows // 2, HIDDEN), jnp.uint32),
        mesh=mesh,
        compiler_params=pltpu.CompilerParams(needs_layout_passes=False),
        scratch_types=[
            pltpu.VMEM((n_chunks, CHUNK), jnp.int32),
            pltpu.VMEM((2 * GROUP, HIDDEN), jnp.float32),
            pltpu.VMEM((2 * PH, HIDDEN), jnp.uint32),
            pltpu.SemaphoreType.DMA,
            pltpu.SemaphoreType.DMA,
        ],
    )
    def k(table_hbm, idx_hbm, out_hbm, idx_v, rows_v, pk_v, gsem, wsem):
        wid = lax.axis_index("s") * NC + lax.axis_index("c")
        base_pk = wid * (per_w // 2)

        pltpu.sync_copy(idx_hbm.at[wid], idx_v)

        def fire_group(g, half):
            for t in range(K):
                pltpu.async_copy(
                    table_hbm.at[idx_v.at[g * K + t]],
                    rows_v.at[pl.ds(half * GROUP + t * CHUNK, CHUNK)],
                    gsem,
                )

        def drain_gathers():
            for _ in range(K):
                pltpu.make_async_copy(
                    table_hbm.at[idx_v.at[0]], rows_v.at[pl.ds(0, CHUNK)], gsem
                ).wait()

        def convert(half):
            fb = half * GROUP
            pb = half * PH

            @plsc.parallel_loop(0, PH, 1, unroll=4)
            def _(l):
                for j in range(HIDDEN // 16):
                    ai = plsc.bitcast(rows_v[fb + l, pl.ds(j * 16, 16)],
                                      jnp.uint32)
                    bi = plsc.bitcast(rows_v[fb + PH + l, pl.ds(j * 16, 16)],
                                      jnp.uint32)
                    word = (ai >> 16) | (bi & jnp.uint32(0xFFFF0000))
                    pk_v[pb + l, pl.ds(j * 16, 16)] = word

        def fire_write(g, half):
            pltpu.async_copy(
                pk_v.at[pl.ds(half * PH, PH)],
                out_hbm.at[pl.ds(base_pk + g * PH, PH)],
                wsem,
            )

        def drain_write():
            pltpu.make_async_copy(
                pk_v.at[pl.ds(0, PH)], out_hbm.at[pl.ds(base_pk, PH)], wsem
            ).wait()

        fire_group(0, 0)
        fire_group(1, 1)
        drain_gathers()
        convert(0)
        fire_write(0, 0)
        fire_group(2, 0)
        drain_gathers()
        convert(1)
        fire_write(1, 1)

        def body(g, _):
            half = g % 2
            fire_group(g + 1, 1 - half)
            drain_gathers()
            drain_write()
            convert(half)
            fire_write(g, half)
            return 0

        lax.fori_loop(2, n_groups - 1, body, 0)

        gl = n_groups - 1
        drain_gathers()
        drain_write()
        convert(gl % 2)
        fire_write(gl, gl % 2)
        drain_write()
        drain_write()

    return k(word_emb, idx3)


def _tc_ln_body(tt_ref, pk_ref, pos_ref, td_ref, gm_ref, bt_ref, o_ref):
    bb, s, h = o_ref.shape
    rows = bb * s
    w = pk_ref[...]
    lo = lax.bitcast_convert_type(w << 16, jnp.float32)
    hi = lax.bitcast_convert_type(w & jnp.uint32(0xFFFF0000), jnp.float32)
    nstrip = (rows // 2) // PH
    x = jnp.stack(
        [lo.reshape(nstrip, PH, h), hi.reshape(nstrip, PH, h)], axis=1
    ).reshape(bb, s, h)
    t = tt_ref[...].astype(jnp.float32)[..., None]
    x = x + pos_ref[...][None, :, :] + t * td_ref[...][None, :, :]
    x2 = x.reshape(rows, h)
    ones = jnp.full((h, h), 1.0 / h, jnp.bfloat16)
    xb = x2.astype(jnp.bfloat16)
    mean = lax.dot_general(xb, ones, (((1,), (0,)), ((), ())),
                           preferred_element_type=jnp.float32)
    m2 = lax.dot_general(xb * xb, ones, (((1,), (0,)), ((), ())),
                         preferred_element_type=jnp.float32)
    var = m2 - mean * mean
    scale = lax.rsqrt(var + EPS) * gm_ref[...].reshape(1, h)
    o_ref[...] = ((x2 - mean) * scale).reshape(bb, s, h) + bt_ref[...][None, :, :]


def _tc_ln(tt, pk, pos, td, gm, bt, prev, off, B, S):
    H = HIDDEN
    Gh = (2 * pk.shape[0]) // (BB * S)
    pkb = BB * S // 2
    in_specs = [
        pl.BlockSpec((BB, S), lambda i, off=off: (i + off, 0)),
        pl.BlockSpec((pkb, H), lambda i: (i, 0)),
        pl.BlockSpec((S, H), lambda i: (0, 0)),
        pl.BlockSpec((1, H), lambda i: (0, 0)),
        pl.BlockSpec((1, H), lambda i: (0, 0)),
        pl.BlockSpec((1, H), lambda i: (0, 0)),
    ]
    args = [tt, pk, pos, td, gm, bt]
    aliases = {}
    if prev is not None:
        in_specs.append(pl.BlockSpec(memory_space=pl.ANY))
        args.append(prev)
        aliases = {6: 0}

    def body(*refs):
        _tc_ln_body(*refs[:6], refs[-1])

    return pl.pallas_call(
        body,
        out_shape=jax.ShapeDtypeStruct((B, S, H), jnp.float32),
        grid=(Gh,),
        in_specs=in_specs,
        out_specs=pl.BlockSpec((BB, S, H), lambda i, off=off: (i + off, 0, 0)),
        input_output_aliases=aliases,
    )(*args)


def kernel(input_ids, token_type_ids, word_emb, pos_emb, type_emb, ln_gamma, ln_beta):
    B, S = input_ids.shape
    H = HIDDEN
    n = B * S
    Bh = B // SLICES
    nh = n // SLICES
    assert nh % (NW * CHUNK) == 0 and Bh % BB == 0 and (BB * S) % GROUP == 0
    n_chunks = nh // (NW * CHUNK)
    ids = input_ids.reshape(SLICES, NW, n_chunks, CHUNK).astype(jnp.int32)
    tt = token_type_ids.astype(jnp.int32)

    pos = pos_emb[:S] + type_emb[0][None, :]
    td = (type_emb[1] - type_emb[0])[None, :]
    gm = ln_gamma[None, :]
    bt = ln_beta[None, :]

    pks = [_sc_gather_pack(word_emb, ids[s]) for s in range(SLICES)]
    out = None
    for s in range(SLICES):
        out = _tc_ln(tt, pks[s], pos, td, gm, bt, out, s * (Bh // BB), B, S)
    return out
